# Initial kernel scaffold; baseline (speedup 1.0000x reference)
import jax
import jax.numpy as jnp
from jax.experimental import pallas as pl


def kernel(x, rule_feat_idxs, rule_threshs, rule_signs, rule_masks, premise_params, consequent_params, attention_weights, interaction_pairs):
    raise NotImplementedError("write your pallas kernel here")


# speedup vs baseline: 4.8367x; 4.8367x over previous
import functools

import jax
import jax.numpy as jnp
from jax.experimental import pallas as pl


def _tree_anfis_block(x_ref, aw_ref, idx_ref, a_ref, c_ref, u_ref,
                      ct1_ref, ct2_ref, ct3_ref, c4_ref, ip_ref, out_ref,
                      *, L, F, R):
    xa = x_ref[...] * aw_ref[...]
    xsq = xa * xa

    iop = jax.lax.broadcasted_iota(jnp.int32, (F, ip_ref.shape[1]), 0)
    p1 = (iop == ip_ref[0:1, :]).astype(jnp.float32)
    p2 = (iop == ip_ref[1:2, :]).astype(jnp.float32)
    inter = (jnp.dot(xa, p1, preferred_element_type=jnp.float32)
             * jnp.dot(xa, p2, preferred_element_type=jnp.float32))

    ro = (jnp.dot(xa, ct1_ref[...], preferred_element_type=jnp.float32)
          + jnp.dot(xsq, ct2_ref[...], preferred_element_type=jnp.float32)
          + jnp.dot(inter, ct3_ref[...], preferred_element_type=jnp.float32)
          + c4_ref[...])

    io = jax.lax.broadcasted_iota(jnp.int32, (F, R), 0)
    acc_n = jnp.ones_like(ro)
    acc_d = jnp.ones_like(ro)
    for l in range(L):
        oh = (io == idx_ref[l:l + 1, :]).astype(jnp.float32)
        sel = jnp.dot(xa, oh, preferred_element_type=jnp.float32)
        e = jnp.exp(a_ref[l:l + 1, :] * sel + c_ref[l:l + 1, :])
        acc_d = acc_d * (1.0 + e)
        acc_n = acc_n * (1.0 + e * u_ref[l:l + 1, :])
    fs = acc_n / acc_d

    s0 = jnp.sum(fs, axis=1, keepdims=True)
    s1 = jnp.sum(fs * ro, axis=1, keepdims=True)
    out_ref[...] = s1 / (s0 + 1e-8)


def kernel(x, rule_feat_idxs, rule_threshs, rule_signs, rule_masks,
           premise_params, consequent_params, attention_weights,
           interaction_pairs):
    B, F = x.shape
    R, L = rule_feat_idxs.shape
    P = interaction_pairs.shape[0]

    bs = premise_params[:, None] * rule_signs
    a_lr = (-bs).T
    c_lr = (bs * rule_threshs).T
    u_lr = (1.0 - rule_masks).T
    idx_lr = rule_feat_idxs.T.astype(jnp.int32)

    def pad8(m):
        return jnp.pad(m, ((0, 8 - m.shape[0]), (0, 0)))

    idx_lr = pad8(idx_lr)
    a_lr = pad8(a_lr)
    c_lr = pad8(c_lr)
    u_lr = pad8(u_lr)

    ct1 = consequent_params[:, 0:F].T
    ct2 = consequent_params[:, F:2 * F].T
    ct3 = consequent_params[:, 2 * F:2 * F + P].T
    c4 = consequent_params[:, 2 * F + P:].T
    aw = attention_weights[None, :]
    ip = jnp.pad(interaction_pairs.T.astype(jnp.int32), ((0, 6), (0, 0)))

    BB = 512
    grid = (B // BB,)

    def bspec(shape, imap):
        return pl.BlockSpec(shape, imap)

    body = functools.partial(_tree_anfis_block, L=L, F=F, R=R)
    y = pl.pallas_call(
        body,
        grid=grid,
        in_specs=[
            bspec((BB, F), lambda i: (i, 0)),
            bspec((1, F), lambda i: (0, 0)),
            bspec((8, R), lambda i: (0, 0)),
            bspec((8, R), lambda i: (0, 0)),
            bspec((8, R), lambda i: (0, 0)),
            bspec((8, R), lambda i: (0, 0)),
            bspec((F, R), lambda i: (0, 0)),
            bspec((F, R), lambda i: (0, 0)),
            bspec((P, R), lambda i: (0, 0)),
            bspec((1, R), lambda i: (0, 0)),
            bspec((8, P), lambda i: (0, 0)),
        ],
        out_specs=bspec((BB, 1), lambda i: (i, 0)),
        out_shape=jax.ShapeDtypeStruct((B, 1), jnp.float32),
    )(x, aw, idx_lr, a_lr, c_lr, u_lr, ct1, ct2, ct3, c4, ip)
    return y
